# Initial kernel scaffold; baseline (speedup 1.0000x reference)
import jax
import jax.numpy as jnp
from jax.experimental import pallas as pl


def kernel(q_full, concept_w, cls_w, cls_b):
    raise NotImplementedError("write your pallas kernel here")


# speedup vs baseline: 37.5672x; 37.5672x over previous
import jax
import jax.numpy as jnp
from jax import lax
from jax.experimental import pallas as pl

_B = 8
_SP = 196
_D = 768
_C = 512
_K = 16
_NCLS = 1000


def _body(q_ref, cwt_ref, clswt_ref, clsb_ref, y_ref):
    qb = q_ref[0]
    qk = jnp.dot(qb, cwt_ref[...], preferred_element_type=jnp.float32)

    iota_s = lax.broadcasted_iota(jnp.int32, (_SP, _C), 0)
    w = qk
    acc = jnp.zeros((_SP, _C), jnp.float32)
    denom = jnp.zeros((1, _C), jnp.float32)
    m0 = None
    for k in range(_K):
        m = jnp.max(w, axis=0, keepdims=True)
        ismax = w == m
        pos = jnp.min(jnp.where(ismax, iota_s, _SP), axis=0, keepdims=True)
        onehot = iota_s == pos
        if k == 0:
            m0 = m
        e = jnp.exp(m - m0)
        acc = acc + jnp.where(onehot, e, 0.0)
        denom = denom + e
        w = jnp.where(onehot, -jnp.inf, w)

    wb = jnp.sum(acc / denom, axis=1, keepdims=True) * (1.0 / _C)
    h = jnp.sum(wb * qb, axis=0, keepdims=True)
    h = jnp.maximum(h, 0.0)
    y = jnp.dot(h, clswt_ref[...], preferred_element_type=jnp.float32)
    y_ref[0] = y + clsb_ref[...]


def kernel(q_full, concept_w, cls_w, cls_b):
    q3 = q_full[:, 1:, :]
    cwt = concept_w.T
    clswt = cls_w.T
    clsb = cls_b.reshape(1, _NCLS)

    return pl.pallas_call(
        _body,
        grid=(_B,),
        in_specs=[
            pl.BlockSpec((1, _SP, _D), lambda b: (b, 0, 0)),
            pl.BlockSpec((_D, _C), lambda b: (0, 0)),
            pl.BlockSpec((_D, _NCLS), lambda b: (0, 0)),
            pl.BlockSpec((1, _NCLS), lambda b: (0, 0)),
        ],
        out_specs=pl.BlockSpec((1, 1, _NCLS), lambda b: (b, 0, 0)),
        out_shape=jax.ShapeDtypeStruct((_B, 1, _NCLS), jnp.float32),
    )(q3, cwt, clswt, clsb).reshape(_B, _NCLS)


# speedup vs baseline: 44.6390x; 1.1882x over previous
import jax
import jax.numpy as jnp
from jax import lax
from jax.experimental import pallas as pl

_B = 8
_S = 197
_D = 768
_C = 512
_K = 16
_NCLS = 1000
_NEG = float("-inf")


def _body(q_ref, cw_ref, clsw_ref, clsb_ref, y_ref):
    qb = q_ref[0]
    qk = lax.dot_general(qb, cw_ref[...], (((1,), (1,)), ((), ())),
                         preferred_element_type=jnp.float32)

    iota_s = lax.broadcasted_iota(jnp.int32, (_S, _C), 0)
    qk0 = jnp.where(iota_s == 0, _NEG, qk)

    w = qk0
    m0 = None
    for k in range(_K):
        m = jnp.max(w, axis=0, keepdims=True)
        cand = jnp.where(w == m, iota_s, _S)
        pos = jnp.min(cand, axis=0, keepdims=True)
        if k == 0:
            m0 = m
        w = jnp.where(cand == pos, _NEG, w)

    e = jnp.exp(qk0 - m0)
    a = jnp.where(w == _NEG, e, 0.0)
    denom = jnp.sum(a, axis=0, keepdims=True)
    wb = jnp.sum(a / denom, axis=1, keepdims=True) * (1.0 / _C)
    h = jnp.sum(wb * qb, axis=0, keepdims=True)
    h = jnp.maximum(h, 0.0)
    y = lax.dot_general(h, clsw_ref[...], (((1,), (1,)), ((), ())),
                        preferred_element_type=jnp.float32)
    y_ref[0] = y + clsb_ref[...]


def kernel(q_full, concept_w, cls_w, cls_b):
    clsb = cls_b.reshape(1, _NCLS)
    return pl.pallas_call(
        _body,
        grid=(_B,),
        in_specs=[
            pl.BlockSpec((1, _S, _D), lambda b: (b, 0, 0)),
            pl.BlockSpec((_C, _D), lambda b: (0, 0)),
            pl.BlockSpec((_NCLS, _D), lambda b: (0, 0)),
            pl.BlockSpec((1, _NCLS), lambda b: (0, 0)),
        ],
        out_specs=pl.BlockSpec((1, 1, _NCLS), lambda b: (b, 0, 0)),
        out_shape=jax.ShapeDtypeStruct((_B, 1, _NCLS), jnp.float32),
    )(q_full, concept_w, cls_w, clsb).reshape(_B, _NCLS)


# speedup vs baseline: 50.1434x; 1.1233x over previous
import jax
import jax.numpy as jnp
from jax import lax
from jax.experimental import pallas as pl

_B = 8
_S = 197
_D = 768
_C = 512
_K = 16
_NCLS = 1000
_NEG = float("-inf")


def _body(q_ref, cw_ref, clsw_ref, clsb_ref, y_ref):
    qb = q_ref[0]
    qk = lax.dot_general(qb, cw_ref[...], (((1,), (1,)), ((), ())),
                         preferred_element_type=jnp.float32)

    iota_s = lax.broadcasted_iota(jnp.int32, (_S, _C), 0)
    qk0 = jnp.where(iota_s == 0, _NEG, qk)

    w = qk0
    m0 = jnp.max(qk0, axis=0, keepdims=True)
    for k in range(_K):
        pos = jnp.argmax(w, axis=0)
        w = jnp.where(iota_s == pos[None, :], _NEG, w)

    e = jnp.exp(qk0 - m0)
    a = jnp.where(w == _NEG, e, 0.0)
    denom = jnp.sum(a, axis=0, keepdims=True)
    wb = jnp.sum(a / denom, axis=1, keepdims=True) * (1.0 / _C)
    h = jnp.sum(wb * qb, axis=0, keepdims=True)
    h = jnp.maximum(h, 0.0)
    y = lax.dot_general(h, clsw_ref[...], (((1,), (1,)), ((), ())),
                        preferred_element_type=jnp.float32)
    y_ref[0] = y + clsb_ref[...]


def kernel(q_full, concept_w, cls_w, cls_b):
    clsb = cls_b.reshape(1, _NCLS)
    return pl.pallas_call(
        _body,
        grid=(_B,),
        in_specs=[
            pl.BlockSpec((1, _S, _D), lambda b: (b, 0, 0)),
            pl.BlockSpec((_C, _D), lambda b: (0, 0)),
            pl.BlockSpec((_NCLS, _D), lambda b: (0, 0)),
            pl.BlockSpec((1, _NCLS), lambda b: (0, 0)),
        ],
        out_specs=pl.BlockSpec((1, 1, _NCLS), lambda b: (b, 0, 0)),
        out_shape=jax.ShapeDtypeStruct((_B, 1, _NCLS), jnp.float32),
    )(q_full, concept_w, cls_w, clsb).reshape(_B, _NCLS)


# speedup vs baseline: 58.8174x; 1.1730x over previous
import jax
import jax.numpy as jnp
from jax import lax
from jax.experimental import pallas as pl
from jax.experimental.pallas import tpu as pltpu

_B = 8
_S = 197
_D = 768
_C = 512
_K = 16
_NCLS = 1000
_NEG = float("-inf")


def _body(q_ref, cw_ref, clsw_ref, clsb_ref, y_ref, h_ref):
    b = pl.program_id(0)
    qb = q_ref[0]
    qk = lax.dot_general(qb, cw_ref[...], (((1,), (1,)), ((), ())),
                         preferred_element_type=jnp.float32)

    iota_s = lax.broadcasted_iota(jnp.int32, (_S, _C), 0)
    qk0 = jnp.where(iota_s == 0, _NEG, qk)

    w = qk0
    m0 = jnp.max(qk0, axis=0, keepdims=True)
    for _ in range(_K):
        pos = jnp.argmax(w, axis=0)
        w = jnp.where(iota_s == pos[None, :], _NEG, w)

    e = jnp.exp(qk0 - m0)
    a = jnp.where(w == _NEG, e, 0.0)
    denom = jnp.sum(a, axis=0, keepdims=True)
    wb = jnp.sum(a / denom, axis=1, keepdims=True) * (1.0 / _C)
    h = jnp.sum(wb * qb, axis=0, keepdims=True)
    h = jnp.maximum(h, 0.0)

    iota_b = lax.broadcasted_iota(jnp.int32, (_B, _D), 0)
    hb = jnp.where(iota_b == b, h, 0.0)
    h_ref[...] = jnp.where(b == 0, hb, h_ref[...] + hb)

    @pl.when(b == _B - 1)
    def _classifier():
        y = lax.dot_general(h_ref[...], clsw_ref[...], (((1,), (1,)), ((), ())),
                            preferred_element_type=jnp.float32)
        y_ref[...] = y + clsb_ref[...]


def kernel(q_full, concept_w, cls_w, cls_b):
    clsb = cls_b.reshape(1, _NCLS)
    return pl.pallas_call(
        _body,
        grid=(_B,),
        in_specs=[
            pl.BlockSpec((1, _S, _D), lambda b: (b, 0, 0)),
            pl.BlockSpec((_C, _D), lambda b: (0, 0)),
            pl.BlockSpec((_NCLS, _D), lambda b: (0, 0)),
            pl.BlockSpec((1, _NCLS), lambda b: (0, 0)),
        ],
        out_specs=pl.BlockSpec((_B, _NCLS), lambda b: (0, 0)),
        out_shape=jax.ShapeDtypeStruct((_B, _NCLS), jnp.float32),
        scratch_shapes=[pltpu.VMEM((_B, _D), jnp.float32)],
    )(q_full, concept_w, cls_w, clsb)
